# Initial kernel scaffold; baseline (speedup 1.0000x reference)
import jax
import jax.numpy as jnp
from jax.experimental import pallas as pl


def kernel(x, edge_index, edge_weight, h):
    raise NotImplementedError("write your pallas kernel here")


# speedup vs baseline: 39.4654x; 39.4654x over previous
import functools

import jax
import jax.numpy as jnp
from jax import lax
from jax.experimental import pallas as pl
from jax.experimental.pallas import tpu as pltpu
from jax.experimental.pallas import tpu_sc as plsc

TT = 11
NI = 8
PP = 4
N = 50000
NPAD = 50176
E = 800000
EPAD = 819200
NW = 32
EC = EPAD // NW
BB = 1280
NB = EC // BB
SB = BB // 128
NSL = NPAD // 16


def _hop_body(v_hbm, src_hbm, dst_hbm, w_hbm, out_hbm,
              v_v, src_v, dst_v, w_v, msgs_v, z_v, o_v, acc_sh):
    cid = lax.axis_index("c")
    sid = lax.axis_index("s")
    wid = cid * 16 + sid
    sl_off = sid * NSL

    def zinit(i, c):
        z_v[pl.ds(i * 16, 16)] = jnp.zeros((16,), jnp.float32)
        return c

    lax.fori_loop(0, NSL // 16, zinit, 0)

    def tstep(t, c):
        pltpu.sync_copy(z_v, acc_sh.at[pl.ds(sl_off, NSL)])
        pltpu.sync_copy(v_hbm.at[pl.ds(t * NPAD, NPAD)], v_v)
        plsc.subcore_barrier()

        def block(b, c2):
            blk = (t * NW + wid) * NB + b
            eoff = blk * BB
            pltpu.sync_copy(src_hbm.at[pl.ds(eoff, BB)], src_v)
            pltpu.sync_copy(w_hbm.at[pl.ds(eoff, BB)], w_v)
            pltpu.sync_copy(dst_hbm.at[blk], dst_v)

            def inner(j, c3):
                r = lax.shift_right_logical(j, 3)
                col = lax.shift_left(lax.bitwise_and(j, 7), 4)
                idx = src_v[pl.ds(j * 16, 16)]
                vals = plsc.load_gather(v_v, [idx])
                msgs_v[r, pl.ds(col, 16)] = w_v[pl.ds(j * 16, 16)] * vals
                return c3

            lax.fori_loop(0, BB // 16, inner, 0)

            def scat(sj, c3):
                pltpu.sync_copy(msgs_v.at[sj], acc_sh.at[dst_v.at[sj]],
                                add=True)
                return c3

            lax.fori_loop(0, SB, scat, 0)
            return c2

        lax.fori_loop(0, NB, block, 0)
        plsc.subcore_barrier()
        pltpu.sync_copy(acc_sh.at[pl.ds(sl_off, NSL)], o_v)
        pltpu.sync_copy(o_v,
                        out_hbm.at[pl.ds((t * 2 + cid) * NPAD + sl_off, NSL)])
        plsc.subcore_barrier()
        return c

    lax.fori_loop(0, TT, tstep, 0)


_HOP = functools.partial(
    pl.kernel,
    out_type=jax.ShapeDtypeStruct((TT * 2 * NPAD,), jnp.float32),
    mesh=plsc.VectorSubcoreMesh(core_axis_name="c", subcore_axis_name="s"),
    compiler_params=pltpu.CompilerParams(needs_layout_passes=False),
    scratch_types=[
        pltpu.VMEM((NPAD,), jnp.float32),
        pltpu.VMEM((BB,), jnp.int32),
        pltpu.VMEM((SB, 128), jnp.int32),
        pltpu.VMEM((BB,), jnp.float32),
        pltpu.VMEM((SB, 128), jnp.float32),
        pltpu.VMEM((NSL,), jnp.float32),
        pltpu.VMEM((NSL,), jnp.float32),
        pltpu.VMEM_SHARED((NPAD,), jnp.float32),
    ],
)(_hop_body)


def _tc_add(p):
    def body(p_ref, o_ref):
        o_ref[...] = (p_ref[:, 0, :] + p_ref[:, 1, :]).reshape(TT * NPAD)

    return pl.pallas_call(
        body,
        out_shape=jax.ShapeDtypeStruct((TT * NPAD,), jnp.float32),
    )(p)


def _tc_combine(h, x2, y1, p2):
    def body(h_ref, x_ref, y1_ref, p2_ref, o_ref):
        for i in range(NI):
            acc = None
            for p in range(PP):
                t = i + p
                y2v = p2_ref[t, 0:1, :] + p2_ref[t, 1:2, :]
                term = (h_ref[0, p] * x_ref[t:t + 1, :]
                        + h_ref[1, p] * y1_ref[t:t + 1, :]
                        + h_ref[2, p] * y2v)
                acc = term if acc is None else acc + term
            o_ref[i:i + 1, :] = jnp.maximum(acc, 0.0)

    return pl.pallas_call(
        body,
        out_shape=jax.ShapeDtypeStruct((NI, NPAD), jnp.float32),
        in_specs=[
            pl.BlockSpec(memory_space=pltpu.SMEM),
            pl.BlockSpec(memory_space=pltpu.VMEM),
            pl.BlockSpec(memory_space=pltpu.VMEM),
            pl.BlockSpec(memory_space=pltpu.VMEM),
        ],
    )(h, x2, y1, p2)


def kernel(x, edge_index, edge_weight, h):
    x2 = jnp.pad(x[:TT, :, 0], ((0, 0), (0, NPAD - N)))
    src = edge_index[:TT, 1, :].astype(jnp.int32)
    dst = edge_index[:TT, 0, :].astype(jnp.int32)
    w = edge_weight[:TT]
    pad = EPAD - E
    pad_idx = (jnp.arange(pad, dtype=jnp.int32) * 16) % N
    pad_idx = jnp.broadcast_to(pad_idx, (TT, pad))
    src = jnp.concatenate([src, pad_idx], axis=1).reshape(TT * NW * NB * BB)
    dst = jnp.concatenate([dst, pad_idx], axis=1).reshape(TT * NW * NB, SB, 128)
    w = jnp.concatenate([w, jnp.zeros((TT, pad), w.dtype)], axis=1)
    w = w.reshape(TT * NW * NB * BB)

    p1 = _HOP(x2.reshape(TT * NPAD), src, dst, w)
    y1 = _tc_add(p1.reshape(TT, 2, NPAD))
    p2 = _HOP(y1, src, dst, w)
    outT = _tc_combine(h, x2, y1.reshape(TT, NPAD), p2.reshape(TT, 2, NPAD))
    return outT[:, :N].T


# speedup vs baseline: 48.7045x; 1.2341x over previous
import functools

import jax
import jax.numpy as jnp
from jax import lax
from jax.experimental import pallas as pl
from jax.experimental.pallas import tpu as pltpu
from jax.experimental.pallas import tpu_sc as plsc

T = 12
TT = 11
NI = 8
PP = 4
N = 50000
NPAD = 50176
E = 800000
NW = 32
BB = 1280
RPT = E // BB
NBF = 20
EC = NBF * BB
SB = BB // 128
NSL = NPAD // 16


def _hop_body(vstride, v_hbm, ei_hbm, dst_hbm, w_hbm, out_hbm,
              v_v, src_v, dst_v, w_v, msgs_v, z_v, o_v, acc_sh):
    cid = lax.axis_index("c")
    sid = lax.axis_index("s")
    wid = cid * 16 + sid
    sl_off = sid * NSL
    nb = jnp.where(wid == NW - 1, RPT - (NW - 1) * NBF, NBF)
    ebase = wid * EC

    def zinit(i, c):
        z_v[pl.ds(i * 16, 16)] = jnp.zeros((16,), jnp.float32)
        return c

    lax.fori_loop(0, NSL // 16, zinit, 0)

    def tstep(t, c):
        pltpu.sync_copy(z_v, acc_sh.at[pl.ds(sl_off, NSL)])
        pltpu.sync_copy(v_hbm.at[pl.ds(t * vstride, N)], v_v.at[pl.ds(0, N)])
        plsc.subcore_barrier()

        def block(b, c2):
            eoff = ebase + b * BB
            pltpu.sync_copy(ei_hbm.at[pl.ds((2 * t + 1) * E + eoff, BB)],
                            src_v)
            pltpu.sync_copy(w_hbm.at[pl.ds(t * E + eoff, BB)], w_v)
            pltpu.sync_copy(dst_hbm.at[t * 2 * RPT + wid * NBF + b], dst_v)

            def inner(j, c3):
                r = lax.shift_right_logical(j, 3)
                col = lax.shift_left(lax.bitwise_and(j, 7), 4)
                idx = src_v[pl.ds(j * 16, 16)]
                vals = plsc.load_gather(v_v, [idx])
                msgs_v[r, pl.ds(col, 16)] = w_v[pl.ds(j * 16, 16)] * vals
                return c3

            lax.fori_loop(0, BB // 16, inner, 0)

            def scat(sj, c3):
                pltpu.sync_copy(msgs_v.at[sj], acc_sh.at[dst_v.at[sj]],
                                add=True)
                return c3

            lax.fori_loop(0, SB, scat, 0)
            return c2

        lax.fori_loop(0, nb, block, 0)
        plsc.subcore_barrier()
        pltpu.sync_copy(acc_sh.at[pl.ds(sl_off, NSL)], o_v)
        pltpu.sync_copy(o_v,
                        out_hbm.at[pl.ds((t * 2 + cid) * NPAD + sl_off, NSL)])
        plsc.subcore_barrier()
        return c

    lax.fori_loop(0, TT, tstep, 0)


def _make_hop(vstride):
    return functools.partial(
        pl.kernel,
        out_type=jax.ShapeDtypeStruct((TT * 2 * NPAD,), jnp.float32),
        mesh=plsc.VectorSubcoreMesh(core_axis_name="c", subcore_axis_name="s"),
        compiler_params=pltpu.CompilerParams(needs_layout_passes=False),
        scratch_types=[
            pltpu.VMEM((NPAD,), jnp.float32),
            pltpu.VMEM((BB,), jnp.int32),
            pltpu.VMEM((SB, 128), jnp.int32),
            pltpu.VMEM((BB,), jnp.float32),
            pltpu.VMEM((SB, 128), jnp.float32),
            pltpu.VMEM((NSL,), jnp.float32),
            pltpu.VMEM((NSL,), jnp.float32),
            pltpu.VMEM_SHARED((NPAD,), jnp.float32),
        ],
    )(functools.partial(_hop_body, vstride))


_HOP_X = _make_hop(N)
_HOP_Y = _make_hop(NPAD)


def _tc_add(p):
    def body(p_ref, o_ref):
        o_ref[...] = (p_ref[:, 0, :] + p_ref[:, 1, :]).reshape(TT * NPAD)

    return pl.pallas_call(
        body,
        out_shape=jax.ShapeDtypeStruct((TT * NPAD,), jnp.float32),
    )(p)


def _tc_combine(h, x2, y1, p2):
    def body(h_ref, x_ref, y1_ref, p2_ref, o_ref):
        for i in range(NI):
            acc = None
            for p in range(PP):
                t = i + p
                y2v = p2_ref[t, 0:1, :] + p2_ref[t, 1:2, :]
                term = (h_ref[0, p] * x_ref[t:t + 1, :]
                        + h_ref[1, p] * y1_ref[t:t + 1, :]
                        + h_ref[2, p] * y2v)
                acc = term if acc is None else acc + term
            o_ref[i:i + 1, :] = jnp.maximum(acc, 0.0)

    return pl.pallas_call(
        body,
        out_shape=jax.ShapeDtypeStruct((NI, NPAD), jnp.float32),
        in_specs=[
            pl.BlockSpec(memory_space=pltpu.SMEM),
            pl.BlockSpec(memory_space=pltpu.VMEM),
            pl.BlockSpec(memory_space=pltpu.VMEM),
            pl.BlockSpec(memory_space=pltpu.VMEM),
        ],
    )(h, x2, y1, p2)


def kernel(x, edge_index, edge_weight, h):
    ei_flat = edge_index.astype(jnp.int32).reshape(T * 2 * E)
    dst3 = ei_flat.reshape(T * 2 * RPT, SB, 128)
    w_flat = edge_weight.reshape(T * E)
    x_flat = x.reshape(T * N)

    p1 = _HOP_X(x_flat, ei_flat, dst3, w_flat)
    y1 = _tc_add(p1.reshape(TT, 2, NPAD))
    p2 = _HOP_Y(y1, ei_flat, dst3, w_flat)
    x2 = jnp.pad(x_flat.reshape(T, N)[:TT], ((0, 0), (0, NPAD - N)))
    outT = _tc_combine(h, x2, y1.reshape(TT, NPAD),
                       p2.reshape(TT, 2, NPAD))
    return outT[:, :N].T


# speedup vs baseline: 79.0588x; 1.6232x over previous
import functools

import jax
import jax.numpy as jnp
from jax import lax
from jax.experimental import pallas as pl
from jax.experimental.pallas import tpu as pltpu
from jax.experimental.pallas import tpu_sc as plsc

T = 12
TT = 11
NI = 8
PP = 4
N = 50000
NPAD = 50176
E = 800000
NW = 32
BB = 1280
RPT = E // BB
NBF = 20
EC = NBF * BB
SB = BB // 128
NSL = NPAD // 16


def _hop_body(vstride, v_hbm, ei_hbm, w_hbm, out_hbm,
              v_v, ed_v, w_v, msgs_v, z_v, o_v, acc_sh):
    cid = lax.axis_index("c")
    sid = lax.axis_index("s")
    wid = cid * 16 + sid
    sl_off = sid * NSL
    nb = jnp.where(wid == NW - 1, RPT - (NW - 1) * NBF, NBF)
    ebase = wid * EC

    def zinit(i, c):
        z_v[pl.ds(i * 16, 16)] = jnp.zeros((16,), jnp.float32)
        return c

    lax.fori_loop(0, NSL // 16, zinit, 0)

    def tstep(t, c):
        pltpu.sync_copy(z_v, acc_sh.at[pl.ds(sl_off, NSL)])
        pltpu.sync_copy(v_hbm.at[pl.ds(t * vstride, N)], v_v.at[pl.ds(0, N)])
        plsc.subcore_barrier()

        def block(b, c2):
            eoff = ebase + b * BB
            pltpu.sync_copy(ei_hbm.at[t, :, pl.ds(eoff, BB)], ed_v)
            pltpu.sync_copy(w_hbm.at[pl.ds(t * E + eoff, BB)], w_v)

            def inner(j, c3):
                r = lax.shift_right_logical(j, 3)
                col = lax.shift_left(lax.bitwise_and(j, 7), 4)
                idx = ed_v[1, pl.ds(j * 16, 16)]
                vals = plsc.load_gather(v_v, [idx])
                msgs_v[r, pl.ds(col, 16)] = w_v[pl.ds(j * 16, 16)] * vals
                return c3

            lax.fori_loop(0, BB // 16, inner, 0)

            def scat(sj, c3):
                pltpu.sync_copy(msgs_v.at[sj],
                                acc_sh.at[ed_v.at[0, pl.ds(sj * 128, 128)]],
                                add=True)
                return c3

            lax.fori_loop(0, SB, scat, 0)
            return c2

        lax.fori_loop(0, nb, block, 0)
        plsc.subcore_barrier()
        pltpu.sync_copy(acc_sh.at[pl.ds(sl_off, NSL)], o_v)
        pltpu.sync_copy(o_v,
                        out_hbm.at[pl.ds((t * 2 + cid) * NPAD + sl_off, NSL)])
        plsc.subcore_barrier()
        return c

    lax.fori_loop(0, TT, tstep, 0)


def _make_hop(vstride):
    return functools.partial(
        pl.kernel,
        out_type=jax.ShapeDtypeStruct((TT * 2 * NPAD,), jnp.float32),
        mesh=plsc.VectorSubcoreMesh(core_axis_name="c", subcore_axis_name="s"),
        compiler_params=pltpu.CompilerParams(needs_layout_passes=False),
        scratch_types=[
            pltpu.VMEM((NPAD,), jnp.float32),
            pltpu.VMEM((2, BB), jnp.int32),
            pltpu.VMEM((BB,), jnp.float32),
            pltpu.VMEM((SB, 128), jnp.float32),
            pltpu.VMEM((NSL,), jnp.float32),
            pltpu.VMEM((NSL,), jnp.float32),
            pltpu.VMEM_SHARED((NPAD,), jnp.float32),
        ],
    )(functools.partial(_hop_body, vstride))


_HOP_X = _make_hop(N)
_HOP_Y = _make_hop(NPAD)


def _tc_flatten_w4(w, in_block, row_base):
    def body(x_ref, o_ref):
        for k in range(4):
            o_ref[pl.ds(k * E, E)] = x_ref[row_base + k, :]

    return pl.pallas_call(
        body,
        grid=(1,),
        in_specs=[pl.BlockSpec((8, E), lambda i: (in_block, 0))],
        out_specs=pl.BlockSpec((4 * E,), lambda i: (0,)),
        out_shape=jax.ShapeDtypeStruct((4 * E,), jnp.float32),
    )(w)


def _tc_flatten_w(w):
    return jnp.concatenate([
        _tc_flatten_w4(w, 0, 0),
        _tc_flatten_w4(w, 0, 4),
        _tc_flatten_w4(w, 1, 0),
    ])


def _tc_add(p):
    def body(p_ref, o_ref):
        o_ref[...] = (p_ref[:, 0, :] + p_ref[:, 1, :]).reshape(TT * NPAD)

    return pl.pallas_call(
        body,
        out_shape=jax.ShapeDtypeStruct((TT * NPAD,), jnp.float32),
    )(p)


def _tc_combine(h, x2, y1, p2):
    def body(h_ref, x_ref, y1_ref, p2_ref, o_ref):
        for i in range(NI):
            acc = None
            for p in range(PP):
                t = i + p
                y2v = p2_ref[t, 0:1, :] + p2_ref[t, 1:2, :]
                term = (h_ref[0, p] * x_ref[t:t + 1, :]
                        + h_ref[1, p] * y1_ref[t:t + 1, :]
                        + h_ref[2, p] * y2v)
                acc = term if acc is None else acc + term
            o_ref[i:i + 1, :] = jnp.maximum(acc, 0.0)

    return pl.pallas_call(
        body,
        out_shape=jax.ShapeDtypeStruct((NI, NPAD), jnp.float32),
        in_specs=[
            pl.BlockSpec(memory_space=pltpu.SMEM),
            pl.BlockSpec(memory_space=pltpu.VMEM),
            pl.BlockSpec(memory_space=pltpu.VMEM),
            pl.BlockSpec(memory_space=pltpu.VMEM),
        ],
    )(h, x2, y1, p2)


def kernel(x, edge_index, edge_weight, h):
    ei = edge_index.astype(jnp.int32)
    w_flat = _tc_flatten_w(edge_weight)
    x_flat = x.reshape(T * N)

    p1 = _HOP_X(x_flat, ei, w_flat)
    y1 = _tc_add(p1.reshape(TT, 2, NPAD))
    p2 = _HOP_Y(y1, ei, w_flat)
    x2 = jnp.pad(x_flat.reshape(T, N)[:TT], ((0, 0), (0, NPAD - N)))
    outT = _tc_combine(h, x2, y1.reshape(TT, NPAD),
                       p2.reshape(TT, 2, NPAD))
    return outT[:, :N].T


# speedup vs baseline: 183.1482x; 2.3166x over previous
import functools

import jax
import jax.numpy as jnp
from jax import lax
from jax.experimental import pallas as pl
from jax.experimental.pallas import tpu as pltpu
from jax.experimental.pallas import tpu_sc as plsc

T = 12
TT = 11
NI = 8
PP = 4
N = 50000
NPAD = 50176
E = 800000
NW = 32
BB = 1280
RPT = E // BB
NBF = 20
EC = NBF * BB
SB = BB // 128
NSL = NPAD // 16


def _hop_body(vstride, v_hbm, ei_hbm, w_hbm, out_hbm,
              v_v, ed0, ed1, ed2, w0, w1, w2, m0, m1, m2, z_v, o_v, acc_sh,
              sem_in, sem_s0, sem_s1, sem_s2):
    sems = (sem_s0, sem_s1, sem_s2)
    eds = (ed0, ed1, ed2)
    ws = (w0, w1, w2)
    msgs = (m0, m1, m2)
    cid = lax.axis_index("c")
    sid = lax.axis_index("s")
    wid = cid * 16 + sid
    sl_off = sid * NSL
    nb = jnp.where(wid == NW - 1, RPT - (NW - 1) * NBF, NBF)
    ebase = wid * EC
    ngroups = (NBF + 2 + 2) // 3 + 1

    def zinit(i, c):
        z_v[pl.ds(i * 16, 16)] = jnp.zeros((16,), jnp.float32)
        return c

    lax.fori_loop(0, NSL // 16, zinit, 0)

    def in_ei(t, b, s):
        return pltpu.make_async_copy(
            ei_hbm.at[t, :, pl.ds(ebase + b * BB, BB)], eds[s], sem_in)

    def in_w(t, b, s):
        return pltpu.make_async_copy(
            w_hbm.at[pl.ds(t * E + ebase + b * BB, BB)], ws[s], sem_in)

    def sc_stream(s, sj):
        return pltpu.make_async_copy(
            msgs[s].at[sj],
            acc_sh.at[eds[s].at[0, pl.ds(sj * 128, 128)]], sems[s])

    def tstep(t, c):
        pltpu.sync_copy(z_v, acc_sh.at[pl.ds(sl_off, NSL)])
        in_ei(t, 0, 0).start()
        in_w(t, 0, 0).start()
        pltpu.sync_copy(v_hbm.at[pl.ds(t * vstride, N)], v_v.at[pl.ds(0, N)])
        plsc.subcore_barrier()

        def group(g, c2):
            for sub in range(3):
                b = g * 3 + sub
                sn = (sub + 1) % 3

                @pl.when(jnp.logical_and(b >= 2, b - 2 < nb))
                def _():
                    for sj in range(SB):
                        sc_stream(sn, sj).wait()

                @pl.when(b + 1 < nb)
                def _():
                    in_ei(t, b + 1, sn).start()
                    in_w(t, b + 1, sn).start()

                @pl.when(b < nb)
                def _():
                    in_ei(t, b, sub).wait()
                    in_w(t, b, sub).wait()

                    def inner(j, c3):
                        r = lax.shift_right_logical(j, 3)
                        col = lax.shift_left(lax.bitwise_and(j, 7), 4)
                        idx = eds[sub][1, pl.ds(j * 16, 16)]
                        vals = plsc.load_gather(v_v, [idx])
                        msgs[sub][r, pl.ds(col, 16)] = (
                            ws[sub][pl.ds(j * 16, 16)] * vals)
                        return c3

                    lax.fori_loop(0, BB // 16, inner, 0)
                    for sj in range(SB):
                        pltpu.async_copy(
                            msgs[sub].at[sj],
                            acc_sh.at[eds[sub].at[0, pl.ds(sj * 128, 128)]],
                            sems[sub], add=True)
            return c2

        lax.fori_loop(0, ngroups, group, 0)
        plsc.subcore_barrier()
        pltpu.sync_copy(acc_sh.at[pl.ds(sl_off, NSL)], o_v)
        pltpu.sync_copy(o_v,
                        out_hbm.at[pl.ds((t * 2 + cid) * NPAD + sl_off, NSL)])
        plsc.subcore_barrier()
        return c

    lax.fori_loop(0, TT, tstep, 0)


def _make_hop(vstride):
    return functools.partial(
        pl.kernel,
        out_type=jax.ShapeDtypeStruct((TT * 2 * NPAD,), jnp.float32),
        mesh=plsc.VectorSubcoreMesh(core_axis_name="c", subcore_axis_name="s"),
        compiler_params=pltpu.CompilerParams(needs_layout_passes=False),
        scratch_types=[
            pltpu.VMEM((NPAD,), jnp.float32),
            pltpu.VMEM((2, BB), jnp.int32),
            pltpu.VMEM((2, BB), jnp.int32),
            pltpu.VMEM((2, BB), jnp.int32),
            pltpu.VMEM((BB,), jnp.float32),
            pltpu.VMEM((BB,), jnp.float32),
            pltpu.VMEM((BB,), jnp.float32),
            pltpu.VMEM((SB, 128), jnp.float32),
            pltpu.VMEM((SB, 128), jnp.float32),
            pltpu.VMEM((SB, 128), jnp.float32),
            pltpu.VMEM((NSL,), jnp.float32),
            pltpu.VMEM((NSL,), jnp.float32),
            pltpu.VMEM_SHARED((NPAD,), jnp.float32),
            pltpu.SemaphoreType.DMA,
            pltpu.SemaphoreType.DMA,
            pltpu.SemaphoreType.DMA,
            pltpu.SemaphoreType.DMA,
        ],
    )(functools.partial(_hop_body, vstride))


_HOP_X = _make_hop(N)
_HOP_Y = _make_hop(NPAD)


def _tc_flatten_w4(w, in_block, row_base):
    def body(x_ref, o_ref):
        for k in range(4):
            o_ref[pl.ds(k * E, E)] = x_ref[row_base + k, :]

    return pl.pallas_call(
        body,
        grid=(1,),
        in_specs=[pl.BlockSpec((8, E), lambda i: (in_block, 0))],
        out_specs=pl.BlockSpec((4 * E,), lambda i: (0,)),
        out_shape=jax.ShapeDtypeStruct((4 * E,), jnp.float32),
    )(w)


def _tc_flatten_w(w):
    return jnp.concatenate([
        _tc_flatten_w4(w, 0, 0),
        _tc_flatten_w4(w, 0, 4),
        _tc_flatten_w4(w, 1, 0),
    ])


def _tc_add(p):
    def body(p_ref, o_ref):
        o_ref[...] = (p_ref[:, 0, :] + p_ref[:, 1, :]).reshape(TT * NPAD)

    return pl.pallas_call(
        body,
        out_shape=jax.ShapeDtypeStruct((TT * NPAD,), jnp.float32),
    )(p)


def _tc_combine(h, x2, y1, p2):
    def body(h_ref, x_ref, y1_ref, p2_ref, o_ref):
        for i in range(NI):
            acc = None
            for p in range(PP):
                t = i + p
                y2v = p2_ref[t, 0:1, :] + p2_ref[t, 1:2, :]
                term = (h_ref[0, p] * x_ref[t:t + 1, :]
                        + h_ref[1, p] * y1_ref[t:t + 1, :]
                        + h_ref[2, p] * y2v)
                acc = term if acc is None else acc + term
            o_ref[i:i + 1, :] = jnp.maximum(acc, 0.0)

    return pl.pallas_call(
        body,
        out_shape=jax.ShapeDtypeStruct((NI, NPAD), jnp.float32),
        in_specs=[
            pl.BlockSpec(memory_space=pltpu.SMEM),
            pl.BlockSpec(memory_space=pltpu.VMEM),
            pl.BlockSpec(memory_space=pltpu.VMEM),
            pl.BlockSpec(memory_space=pltpu.VMEM),
        ],
    )(h, x2, y1, p2)


def kernel(x, edge_index, edge_weight, h):
    ei = edge_index.astype(jnp.int32)
    w_flat = _tc_flatten_w(edge_weight)
    x_flat = x.reshape(T * N)

    p1 = _HOP_X(x_flat, ei, w_flat)
    y1 = _tc_add(p1.reshape(TT, 2, NPAD))
    p2 = _HOP_Y(y1, ei, w_flat)
    x2 = jnp.pad(x_flat.reshape(T, N)[:TT], ((0, 0), (0, NPAD - N)))
    outT = _tc_combine(h, x2, y1.reshape(TT, NPAD),
                       p2.reshape(TT, 2, NPAD))
    return outT[:, :N].T


# speedup vs baseline: 190.5823x; 1.0406x over previous
import functools

import jax
import jax.numpy as jnp
from jax import lax
from jax.experimental import pallas as pl
from jax.experimental.pallas import tpu as pltpu
from jax.experimental.pallas import tpu_sc as plsc

T = 12
TT = 11
NI = 8
PP = 4
N = 50000
NPAD = 50176
E = 800000
NW = 32
BB = 1280
RPT = E // BB
NBF = 20
EC = NBF * BB
SB = BB // 128
NSL = NPAD // 16


def _hop_body(vstride, v_hbm, ei_hbm, wg0_hbm, wg1_hbm, wg2_hbm, out_hbm,
              v_v, ed0, ed1, ed2, w0, w1, w2, m0, m1, m2,
              z_v, o_v, acc_sh, sem_in, sem_s0, sem_s1, sem_s2):
    wgs = (wg0_hbm, wg1_hbm, wg2_hbm)
    sems = (sem_s0, sem_s1, sem_s2)
    eds = (ed0, ed1, ed2)
    ws = (w0, w1, w2)
    msgs = (m0, m1, m2)
    cid = lax.axis_index("c")
    sid = lax.axis_index("s")
    wid = cid * 16 + sid
    sl_off = sid * NSL
    nb = jnp.where(wid == NW - 1, RPT - (NW - 1) * NBF, NBF)
    ebase = wid * EC
    ngroups = (NBF + 2 + 2) // 3 + 1

    def zinit(i, c):
        z_v[pl.ds(i * 16, 16)] = jnp.zeros((16,), jnp.float32)
        return c

    lax.fori_loop(0, NSL // 16, zinit, 0)

    def in_ei(t, b, s):
        return pltpu.make_async_copy(
            ei_hbm.at[t, :, pl.ds(ebase + b * BB, BB)], eds[s], sem_in)

    def in_w(w_hbm, tt, b, s):
        return pltpu.make_async_copy(
            w_hbm.at[pl.ds(tt * E + ebase + b * BB, BB)], ws[s], sem_in)

    def sc_stream(s, sj):
        return pltpu.make_async_copy(
            msgs[s].at[sj],
            acc_sh.at[eds[s].at[0, pl.ds(sj * 128, 128)]], sems[s])

    def make_tstep(wg, w_hbm):
        def tstep(tt, c):
            t = wg * 4 + tt
            pltpu.sync_copy(z_v, acc_sh.at[pl.ds(sl_off, NSL)])
            in_ei(t, 0, 0).start()
            in_w(w_hbm, tt, 0, 0).start()
            pltpu.sync_copy(v_hbm.at[pl.ds(t * vstride, N)],
                            v_v.at[pl.ds(0, N)])
            plsc.subcore_barrier()

            def group(g, c2):
                for sub in range(3):
                    b = g * 3 + sub
                    sn = (sub + 1) % 3

                    @pl.when(jnp.logical_and(b >= 2, b - 2 < nb))
                    def _():
                        for sj in range(SB):
                            sc_stream(sn, sj).wait()

                    @pl.when(b + 1 < nb)
                    def _():
                        in_ei(t, b + 1, sn).start()
                        in_w(w_hbm, tt, b + 1, sn).start()

                    @pl.when(b < nb)
                    def _():
                        in_ei(t, b, sub).wait()
                        in_w(w_hbm, tt, b, sub).wait()

                        def inner(j, c3):
                            r = lax.shift_right_logical(j, 3)
                            col = lax.shift_left(lax.bitwise_and(j, 7), 4)
                            idx = eds[sub][1, pl.ds(j * 16, 16)]
                            vals = plsc.load_gather(v_v, [idx])
                            msgs[sub][r, pl.ds(col, 16)] = (
                                ws[sub][pl.ds(j * 16, 16)] * vals)
                            return c3

                        lax.fori_loop(0, BB // 16, inner, 0)
                        for sj in range(SB):
                            pltpu.async_copy(
                                msgs[sub].at[sj],
                                acc_sh.at[eds[sub].at[0, pl.ds(sj * 128, 128)]],
                                sems[sub], add=True)
                return c2

            lax.fori_loop(0, ngroups, group, 0)
            plsc.subcore_barrier()
            pltpu.sync_copy(acc_sh.at[pl.ds(sl_off, NSL)], o_v)
            pltpu.sync_copy(
                o_v, out_hbm.at[pl.ds((t * 2 + cid) * NPAD + sl_off, NSL)])
            plsc.subcore_barrier()
            return c

        return tstep

    for wg in range(3):
        lax.fori_loop(0, 4 if wg < 2 else 3, make_tstep(wg, wgs[wg]), 0)


def _make_hop(vstride):
    return functools.partial(
        pl.kernel,
        out_type=jax.ShapeDtypeStruct((TT * 2 * NPAD,), jnp.float32),
        mesh=plsc.VectorSubcoreMesh(core_axis_name="c", subcore_axis_name="s"),
        compiler_params=pltpu.CompilerParams(needs_layout_passes=False),
        scratch_types=[
            pltpu.VMEM((NPAD,), jnp.float32),
            pltpu.VMEM((2, BB), jnp.int32),
            pltpu.VMEM((2, BB), jnp.int32),
            pltpu.VMEM((2, BB), jnp.int32),
            pltpu.VMEM((BB,), jnp.float32),
            pltpu.VMEM((BB,), jnp.float32),
            pltpu.VMEM((BB,), jnp.float32),
            pltpu.VMEM((SB, 128), jnp.float32),
            pltpu.VMEM((SB, 128), jnp.float32),
            pltpu.VMEM((SB, 128), jnp.float32),
            pltpu.VMEM((NSL,), jnp.float32),
            pltpu.VMEM((NSL,), jnp.float32),
            pltpu.VMEM_SHARED((NPAD,), jnp.float32),
            pltpu.SemaphoreType.DMA,
            pltpu.SemaphoreType.DMA,
            pltpu.SemaphoreType.DMA,
            pltpu.SemaphoreType.DMA,
        ],
    )(functools.partial(_hop_body, vstride))


_HOP_X = _make_hop(N)
_HOP_Y = _make_hop(NPAD)


def _tc_flatten_w4(w, in_block, row_base):
    def body(x_ref, o_ref):
        for k in range(4):
            o_ref[pl.ds(k * E, E)] = x_ref[row_base + k, :]

    return pl.pallas_call(
        body,
        grid=(1,),
        in_specs=[pl.BlockSpec((8, E), lambda i: (in_block, 0))],
        out_specs=pl.BlockSpec((4 * E,), lambda i: (0,)),
        out_shape=jax.ShapeDtypeStruct((4 * E,), jnp.float32),
    )(w)


def _tc_flatten_w(w):
    return (_tc_flatten_w4(w, 0, 0),
            _tc_flatten_w4(w, 0, 4),
            _tc_flatten_w4(w, 1, 0))


def _tc_add(p):
    def body(p_ref, o_ref):
        o_ref[...] = (p_ref[:, 0, :] + p_ref[:, 1, :]).reshape(TT * NPAD)

    return pl.pallas_call(
        body,
        out_shape=jax.ShapeDtypeStruct((TT * NPAD,), jnp.float32),
    )(p)


def _tc_combine(h, x2, y1, p2):
    def body(h_ref, x_ref, y1_ref, p2_ref, o_ref):
        for i in range(NI):
            acc = None
            for p in range(PP):
                t = i + p
                y2v = p2_ref[t, 0:1, :] + p2_ref[t, 1:2, :]
                term = (h_ref[0, p] * x_ref[t:t + 1, :]
                        + h_ref[1, p] * y1_ref[t:t + 1, :]
                        + h_ref[2, p] * y2v)
                acc = term if acc is None else acc + term
            o_ref[i:i + 1, :] = jnp.maximum(acc, 0.0)

    return pl.pallas_call(
        body,
        out_shape=jax.ShapeDtypeStruct((NI, NPAD), jnp.float32),
        in_specs=[
            pl.BlockSpec(memory_space=pltpu.SMEM),
            pl.BlockSpec(memory_space=pltpu.VMEM),
            pl.BlockSpec(memory_space=pltpu.VMEM),
            pl.BlockSpec(memory_space=pltpu.VMEM),
        ],
    )(h, x2, y1, p2)


def kernel(x, edge_index, edge_weight, h):
    ei = edge_index.astype(jnp.int32)
    wg0, wg1, wg2 = _tc_flatten_w(edge_weight)
    x_flat = x.reshape(T * N)

    p1 = _HOP_X(x_flat, ei, wg0, wg1, wg2)
    y1 = _tc_add(p1.reshape(TT, 2, NPAD))
    p2 = _HOP_Y(y1, ei, wg0, wg1, wg2)
    x2 = jnp.pad(x_flat.reshape(T, N)[:TT], ((0, 0), (0, NPAD - N)))
    outT = _tc_combine(h, x2, y1.reshape(TT, NPAD),
                       p2.reshape(TT, 2, NPAD))
    return outT[:, :N].T
